# baseline (speedup 1.0000x reference)
import functools

import jax
import jax.numpy as jnp
from jax import lax
from jax.experimental import pallas as pl
from jax.experimental.pallas import tpu as pltpu
from jax.experimental.pallas import tpu_sc as plsc

SCALE_BOUND = 0.11
LIKELIHOOD_BOUND = 1e-09

_MAGIC = 12582912.0
_BIG = 8388608.0
_C1 = 1.5976
_C3 = 0.070565992

_B = 8
_R = 1536
_NW = 32
_RW = (_B * _R) // _NW
_CH = 96
_NCH = _RW // _CH


def _vreg_compute(xv, sv, mv):
    d = xv - mv
    big = jnp.abs(d) >= _BIG
    q = (d + _MAGIC) - _MAGIC
    q = jnp.where(big, d, q)
    out = q + mv
    v = jnp.abs(q)
    sb = jnp.maximum(sv, SCALE_BOUND)
    inv = 1.0 / sb
    zu = (0.5 - v) * inv
    zl = (-0.5 - v) * inv
    fu = zu * (_C1 + _C3 * (zu * zu))
    fl = zl * (_C1 + _C3 * (zl * zl))
    eu = jnp.exp(fu)
    el = jnp.exp(fl)
    lik = (eu - el) / ((1.0 + eu) * (1.0 + el))
    lik = jnp.maximum(lik, LIKELIHOOD_BOUND)
    return out, lik


def _sc_body(x_hbm, s_hbm, m_hbm, out_hbm, lik_hbm, xb, sb, mb, ob, lb):
    wid = lax.axis_index("s") * 2 + lax.axis_index("c")
    row0 = wid * _RW
    b = row0 // _R
    r_in_b = row0 - b * _R

    for k in range(_NCH):
        r0 = r_in_b + k * _CH
        pltpu.sync_copy(x_hbm.at[b, pl.ds(r0, _CH)], xb)
        pltpu.sync_copy(s_hbm.at[b, pl.ds(r0, _CH)], sb)
        pltpu.sync_copy(m_hbm.at[b, pl.ds(r0, _CH)], mb)

        def row(r, carry):
            for j in range(8):
                sl = pl.ds(j * 16, 16)
                o, l = _vreg_compute(xb[r, sl], sb[r, sl], mb[r, sl])
                ob[r, sl] = o
                lb[r, sl] = l
            return carry

        lax.fori_loop(0, _CH, row, 0)

        pltpu.sync_copy(ob, out_hbm.at[b, pl.ds(r0, _CH)])
        pltpu.sync_copy(lb, lik_hbm.at[b, pl.ds(r0, _CH)])


def kernel(x, scales, means):
    shape = x.shape
    r3 = (_B, _R, 128)
    x3 = x.reshape(r3)
    s3 = scales.reshape(r3)
    m3 = means.reshape(r3)

    mesh = plsc.VectorSubcoreMesh(core_axis_name="c", subcore_axis_name="s")
    f = functools.partial(
        pl.kernel,
        mesh=mesh,
        out_type=[
            jax.ShapeDtypeStruct(r3, jnp.float32),
            jax.ShapeDtypeStruct(r3, jnp.float32),
        ],
        scratch_types=[
            pltpu.VMEM((_CH, 128), jnp.float32),
            pltpu.VMEM((_CH, 128), jnp.float32),
            pltpu.VMEM((_CH, 128), jnp.float32),
            pltpu.VMEM((_CH, 128), jnp.float32),
            pltpu.VMEM((_CH, 128), jnp.float32),
        ],
    )(_sc_body)
    out, lik = f(x3, s3, m3)
    return out.reshape(shape), lik.reshape(shape)


# speedup vs baseline: 1.0760x; 1.0760x over previous
import functools

import jax
import jax.numpy as jnp
from jax import lax
from jax.experimental import pallas as pl
from jax.experimental.pallas import tpu as pltpu
from jax.experimental.pallas import tpu_sc as plsc

SCALE_BOUND = 0.11
LIKELIHOOD_BOUND = 1e-09

_MAGIC = 12582912.0
_BIG = 8388608.0
_C1 = 1.5976
_C3 = 0.070565992

_B = 8
_R = 1536
_NW = 32
_RW = (_B * _R) // _NW
_CH = 96
_NCH = _RW // _CH


def _vreg_compute(xv, sv, mv):
    d = xv - mv
    big = jnp.abs(d) >= _BIG
    q = (d + _MAGIC) - _MAGIC
    q = jnp.where(big, d, q)
    out = q + mv
    v = jnp.abs(q)
    sb = jnp.maximum(sv, SCALE_BOUND)
    inv = 1.0 / sb
    zu = (0.5 - v) * inv
    zl = (-0.5 - v) * inv
    fu = zu * (_C1 + _C3 * (zu * zu))
    fl = zl * (_C1 + _C3 * (zl * zl))
    eu = jnp.exp(fu)
    el = jnp.exp(fl)
    lik = (eu - el) / ((1.0 + eu) * (1.0 + el))
    lik = jnp.maximum(lik, LIKELIHOOD_BOUND)
    return out, lik


def _sc_body(x_hbm, s_hbm, m_hbm, out_hbm, lik_hbm,
             xb, sb, mb, ob, lb, si0, si1, so0, so1):
    wid = lax.axis_index("s") * 2 + lax.axis_index("c")
    row0 = wid * _RW
    b = row0 // _R
    r_in_b = row0 - b * _R
    sem_in = (si0, si1)
    sem_out = (so0, so1)

    def start_in(k):
        buf = k % 2
        r0 = r_in_b + k * _CH
        return [
            pltpu.async_copy(x_hbm.at[b, pl.ds(r0, _CH)], xb.at[buf], sem_in[buf]),
            pltpu.async_copy(s_hbm.at[b, pl.ds(r0, _CH)], sb.at[buf], sem_in[buf]),
            pltpu.async_copy(m_hbm.at[b, pl.ds(r0, _CH)], mb.at[buf], sem_in[buf]),
        ]

    hin = [None] * _NCH
    hout = [None] * _NCH
    hin[0] = start_in(0)
    for k in range(_NCH):
        cur = k % 2
        if k + 1 < _NCH:
            hin[k + 1] = start_in(k + 1)
        for h in hin[k]:
            h.wait()
        if k >= 2:
            for h in hout[k - 2]:
                h.wait()

        @plsc.parallel_loop(0, _CH, 1)
        def row(r):
            for j in range(8):
                sl = pl.ds(j * 16, 16)
                o, l = _vreg_compute(xb[cur, r, sl], sb[cur, r, sl], mb[cur, r, sl])
                ob[cur, r, sl] = o
                lb[cur, r, sl] = l

        r0 = r_in_b + k * _CH
        hout[k] = [
            pltpu.async_copy(ob.at[cur], out_hbm.at[b, pl.ds(r0, _CH)], sem_out[cur]),
            pltpu.async_copy(lb.at[cur], lik_hbm.at[b, pl.ds(r0, _CH)], sem_out[cur]),
        ]
    for h in hout[_NCH - 2]:
        h.wait()
    for h in hout[_NCH - 1]:
        h.wait()


def kernel(x, scales, means):
    shape = x.shape
    r3 = (_B, _R, 128)
    x3 = x.reshape(r3)
    s3 = scales.reshape(r3)
    m3 = means.reshape(r3)

    mesh = plsc.VectorSubcoreMesh(core_axis_name="c", subcore_axis_name="s")
    f = functools.partial(
        pl.kernel,
        mesh=mesh,
        out_type=[
            jax.ShapeDtypeStruct(r3, jnp.float32),
            jax.ShapeDtypeStruct(r3, jnp.float32),
        ],
        scratch_types=[
            pltpu.VMEM((2, _CH, 128), jnp.float32),
            pltpu.VMEM((2, _CH, 128), jnp.float32),
            pltpu.VMEM((2, _CH, 128), jnp.float32),
            pltpu.VMEM((2, _CH, 128), jnp.float32),
            pltpu.VMEM((2, _CH, 128), jnp.float32),
            pltpu.SemaphoreType.DMA,
            pltpu.SemaphoreType.DMA,
            pltpu.SemaphoreType.DMA,
            pltpu.SemaphoreType.DMA,
        ],
    )(_sc_body)
    out, lik = f(x3, s3, m3)
    return out.reshape(shape), lik.reshape(shape)


# speedup vs baseline: 1.0771x; 1.0011x over previous
import functools

import jax
import jax.numpy as jnp
from jax import lax
from jax.experimental import pallas as pl
from jax.experimental.pallas import tpu as pltpu
from jax.experimental.pallas import tpu_sc as plsc

SCALE_BOUND = 0.11
LIKELIHOOD_BOUND = 1e-09

_MAGIC = 12582912.0
_BIG = 8388608.0
_C1 = 1.5976
_C3 = 0.070565992

_B = 8
_R = 1536
_NW = 32
_RW = (_B * _R) // _NW
_CH = 96
_NCH = _RW // _CH


def _vreg_compute(xv, sv, mv):
    d = xv - mv
    big = jnp.abs(d) >= _BIG
    q = (d + _MAGIC) - _MAGIC
    q = jnp.where(big, d, q)
    out = q + mv
    v = jnp.abs(q)
    sb = jnp.maximum(sv, SCALE_BOUND)
    inv = 1.0 / sb
    zu = (0.5 - v) * inv
    zl = (-0.5 - v) * inv
    fu = zu * (_C1 + _C3 * (zu * zu))
    fl = zl * (_C1 + _C3 * (zl * zl))
    eu = jnp.exp(fu)
    el = jnp.exp(fl)
    lik = (eu - el) / ((1.0 + eu) * (1.0 + el))
    lik = jnp.maximum(lik, LIKELIHOOD_BOUND)
    return out, lik


def _sc_body(x_hbm, s_hbm, m_hbm, out_hbm, lik_hbm,
             xb, sb, mb, ob, lb, si0, si1, so0, so1):
    wid = lax.axis_index("s") * 2 + lax.axis_index("c")
    row0 = wid * _RW
    b = row0 // _R
    r_in_b = row0 - b * _R
    sem_in = (si0, si1)
    sem_out = (so0, so1)

    def start_in(k):
        buf = k % 2
        r0 = r_in_b + k * _CH
        return [
            pltpu.async_copy(x_hbm.at[b, pl.ds(r0, _CH)], xb.at[buf], sem_in[buf]),
            pltpu.async_copy(s_hbm.at[b, pl.ds(r0, _CH)], sb.at[buf], sem_in[buf]),
            pltpu.async_copy(m_hbm.at[b, pl.ds(r0, _CH)], mb.at[buf], sem_in[buf]),
        ]

    hin = [None] * _NCH
    hout = [None] * _NCH
    hin[0] = start_in(0)
    for k in range(_NCH):
        cur = k % 2
        if k + 1 < _NCH:
            hin[k + 1] = start_in(k + 1)
        for h in hin[k]:
            h.wait()
        if k >= 2:
            for h in hout[k - 2]:
                h.wait()

        @plsc.parallel_loop(0, _CH, 1)
        def row(r):
            sls = [pl.ds(j * 16, 16) for j in range(8)]
            xs = [xb[cur, r, sl] for sl in sls]
            ms = [mb[cur, r, sl] for sl in sls]
            ss = [sb[cur, r, sl] for sl in sls]
            ds = [xv - mv for xv, mv in zip(xs, ms)]
            qs = [(d + _MAGIC) - _MAGIC for d in ds]
            qs = [jnp.where(jnp.abs(d) >= _BIG, d, q) for d, q in zip(ds, qs)]
            outs = [q + mv for q, mv in zip(qs, ms)]
            vs = [jnp.abs(q) for q in qs]
            invs = [1.0 / jnp.maximum(sv, SCALE_BOUND) for sv in ss]
            zus = [(0.5 - v) * inv for v, inv in zip(vs, invs)]
            zls = [(-0.5 - v) * inv for v, inv in zip(vs, invs)]
            fus = [zu * (_C1 + _C3 * (zu * zu)) for zu in zus]
            fls = [zl * (_C1 + _C3 * (zl * zl)) for zl in zls]
            eus = [jnp.exp(fu) for fu in fus]
            els = [jnp.exp(fl) for fl in fls]
            liks = [
                jnp.maximum((eu - el) / ((1.0 + eu) * (1.0 + el)),
                            LIKELIHOOD_BOUND)
                for eu, el in zip(eus, els)
            ]
            for sl, o, l in zip(sls, outs, liks):
                ob[cur, r, sl] = o
                lb[cur, r, sl] = l

        r0 = r_in_b + k * _CH
        hout[k] = [
            pltpu.async_copy(ob.at[cur], out_hbm.at[b, pl.ds(r0, _CH)], sem_out[cur]),
            pltpu.async_copy(lb.at[cur], lik_hbm.at[b, pl.ds(r0, _CH)], sem_out[cur]),
        ]
    for h in hout[_NCH - 2]:
        h.wait()
    for h in hout[_NCH - 1]:
        h.wait()


def kernel(x, scales, means):
    shape = x.shape
    r3 = (_B, _R, 128)
    x3 = x.reshape(r3)
    s3 = scales.reshape(r3)
    m3 = means.reshape(r3)

    mesh = plsc.VectorSubcoreMesh(core_axis_name="c", subcore_axis_name="s")
    f = functools.partial(
        pl.kernel,
        mesh=mesh,
        out_type=[
            jax.ShapeDtypeStruct(r3, jnp.float32),
            jax.ShapeDtypeStruct(r3, jnp.float32),
        ],
        scratch_types=[
            pltpu.VMEM((2, _CH, 128), jnp.float32),
            pltpu.VMEM((2, _CH, 128), jnp.float32),
            pltpu.VMEM((2, _CH, 128), jnp.float32),
            pltpu.VMEM((2, _CH, 128), jnp.float32),
            pltpu.VMEM((2, _CH, 128), jnp.float32),
            pltpu.SemaphoreType.DMA,
            pltpu.SemaphoreType.DMA,
            pltpu.SemaphoreType.DMA,
            pltpu.SemaphoreType.DMA,
        ],
    )(_sc_body)
    out, lik = f(x3, s3, m3)
    return out.reshape(shape), lik.reshape(shape)


# speedup vs baseline: 2.7739x; 2.5754x over previous
import jax
import jax.numpy as jnp
from jax.experimental import pallas as pl

SCALE_BOUND = 0.11
LIKELIHOOD_BOUND = 1e-09

_P = 0.3275911
_A1 = 0.254829592
_A2 = -0.284496736
_A3 = 1.421413741
_A4 = -1.453152027
_A5 = 1.061405429
_INV_SQRT2 = 0.7071067811865476


def _erfc_nonneg(a):
    t = 1.0 / (1.0 + _P * a)
    poly = t * (_A1 + t * (_A2 + t * (_A3 + t * (_A4 + t * _A5))))
    return poly * jnp.exp(-(a * a))


def _body(x_ref, s_ref, m_ref, out_ref, lik_ref):
    x = x_ref[...]
    s = s_ref[...]
    m = m_ref[...]
    q = jnp.round(x - m)
    out_ref[...] = q + m
    v = jnp.abs(q)
    sb = jnp.maximum(s, SCALE_BOUND)
    inv = _INV_SQRT2 / sb
    a = (v + 0.5) * inv
    b = (v - 0.5) * inv
    ea = _erfc_nonneg(a)
    eb_mag = _erfc_nonneg(jnp.abs(b))
    eb = jnp.where(b < 0.0, 2.0 - eb_mag, eb_mag)
    lik = 0.5 * (eb - ea)
    lik_ref[...] = jnp.maximum(lik, LIKELIHOOD_BOUND)


def kernel(x, scales, means):
    shape = x.shape
    b, c, h, w = shape
    r4 = (b, c, (h * w) // 128, 128)
    x4 = x.reshape(r4)
    s4 = scales.reshape(r4)
    m4 = means.reshape(r4)
    bc = 96
    grid = (b, c // bc)
    spec = pl.BlockSpec((1, bc, r4[2], 128), lambda i, j: (i, j, 0, 0))
    out, lik = pl.pallas_call(
        _body,
        grid=grid,
        in_specs=[spec, spec, spec],
        out_specs=[spec, spec],
        out_shape=[
            jax.ShapeDtypeStruct(r4, jnp.float32),
            jax.ShapeDtypeStruct(r4, jnp.float32),
        ],
    )(x4, s4, m4)
    return out.reshape(shape), lik.reshape(shape)


# speedup vs baseline: 3.0054x; 1.0835x over previous
import jax
import jax.numpy as jnp
from jax.experimental import pallas as pl

SCALE_BOUND = 0.11
LIKELIHOOD_BOUND = 1e-09

_P = 0.3275911
_A1 = 0.254829592
_A2 = -0.284496736
_A3 = 1.421413741
_A4 = -1.453152027
_A5 = 1.061405429
_INV_SQRT2 = 0.7071067811865476


def _erfc_nonneg(a):
    t = 1.0 / (1.0 + _P * a)
    poly = t * (_A1 + t * (_A2 + t * (_A3 + t * (_A4 + t * _A5))))
    return poly * jnp.exp(-(a * a))


def _body(x_ref, s_ref, m_ref, out_ref, lik_ref):
    x = x_ref[...]
    s = s_ref[...]
    m = m_ref[...]
    q = jnp.round(x - m)
    out_ref[...] = q + m
    v = jnp.abs(q)
    sb = jnp.maximum(s, SCALE_BOUND)
    inv = _INV_SQRT2 / sb
    a = (v + 0.5) * inv
    b = (v - 0.5) * inv
    ea = _erfc_nonneg(a)
    eb_mag = _erfc_nonneg(jnp.abs(b))
    eb = jnp.where(b < 0.0, 2.0 - eb_mag, eb_mag)
    lik = 0.5 * (eb - ea)
    lik_ref[...] = jnp.maximum(lik, LIKELIHOOD_BOUND)


def kernel(x, scales, means):
    shape = x.shape
    b, c, h, w = shape
    r4 = (b, c, (h * w) // 128, 128)
    x4 = x.reshape(r4)
    s4 = scales.reshape(r4)
    m4 = means.reshape(r4)
    bb, bc = 2, 192
    grid = (b // bb, c // bc)
    spec = pl.BlockSpec((bb, bc, r4[2], 128), lambda i, j: (i, j, 0, 0))
    out, lik = pl.pallas_call(
        _body,
        grid=grid,
        in_specs=[spec, spec, spec],
        out_specs=[spec, spec],
        out_shape=[
            jax.ShapeDtypeStruct(r4, jnp.float32),
            jax.ShapeDtypeStruct(r4, jnp.float32),
        ],
    )(x4, s4, m4)
    return out.reshape(shape), lik.reshape(shape)
